# Initial kernel scaffold; baseline (speedup 1.0000x reference)
import jax
import jax.numpy as jnp
from jax.experimental import pallas as pl
from jax.experimental.pallas import tpu as pltpu

_N_GROUP = 8
_TOPK_GROUP = 4
_TOP_K = 8
_SCALE = 2.5
_NEG = jnp.float32(-1e30)


def _gate_kernel(x_ref, wt_ref, b_ref, idx_ref, w_ref):
    x = x_ref[...]
    wt = wt_ref[...]
    logits = jnp.dot(x, wt, preferred_element_type=jnp.float32)
    scores = jax.nn.sigmoid(logits)
    sfc = scores + b_ref[...]

    bt, e = scores.shape
    spg = e // _N_GROUP
    lane = jax.lax.broadcasted_iota(jnp.int32, (bt, e), 1)
    gid = lane // spg

    gs_cols = []
    for g in range(_N_GROUP):
        in_g = gid == g
        m1 = jnp.max(jnp.where(in_g, sfc, _NEG), axis=1, keepdims=True)
        i1 = jnp.min(jnp.where(in_g & (sfc == m1), lane, e), axis=1,
                     keepdims=True)
        m2 = jnp.max(jnp.where(in_g & (lane != i1), sfc, _NEG), axis=1,
                     keepdims=True)
        gs_cols.append(m1 + m2)
    gs = jnp.concatenate(gs_cols, axis=1)

    lane_g = jax.lax.broadcasted_iota(jnp.int32, (bt, _N_GROUP), 1)
    gmask = jnp.zeros((bt, e), dtype=jnp.bool_)
    t = gs
    for _ in range(_TOPK_GROUP):
        m = jnp.max(t, axis=1, keepdims=True)
        gsel = jnp.min(jnp.where(t == m, lane_g, _N_GROUP), axis=1,
                       keepdims=True)
        gmask = gmask | (gid == gsel)
        t = jnp.where(lane_g == gsel, _NEG, t)

    tmp = jnp.where(gmask, sfc, 0.0)

    col = jax.lax.broadcasted_iota(jnp.int32, (bt, _TOP_K), 1)
    acc_idx = jnp.zeros((bt, _TOP_K), dtype=jnp.int32)
    acc_w = jnp.zeros((bt, _TOP_K), dtype=jnp.float32)
    t = tmp
    for k in range(_TOP_K):
        m = jnp.max(t, axis=1, keepdims=True)
        i = jnp.min(jnp.where(t == m, lane, e), axis=1, keepdims=True)
        onehot = lane == i
        wk = jnp.max(jnp.where(onehot, scores, _NEG), axis=1, keepdims=True)
        acc_idx = jnp.where(col == k, i, acc_idx)
        acc_w = jnp.where(col == k, wk, acc_w)
        t = jnp.where(onehot, _NEG, t)

    denom = jnp.sum(acc_w, axis=1, keepdims=True) + 1e-20
    idx_ref[...] = acc_idx
    w_ref[...] = acc_w / denom * _SCALE


def kernel(hidden_states, weight, e_score_correction_bias):
    bsz, seq, h = hidden_states.shape
    n_experts = weight.shape[0]
    t = bsz * seq
    bt = 256

    x2 = hidden_states.reshape(t, h)
    wt = weight.astype(jnp.float32).T
    b2 = e_score_correction_bias.reshape(1, n_experts).astype(jnp.float32)

    idx, w = pl.pallas_call(
        _gate_kernel,
        grid=(t // bt,),
        in_specs=[
            pl.BlockSpec((bt, h), lambda i: (i, 0)),
            pl.BlockSpec((h, n_experts), lambda i: (0, 0)),
            pl.BlockSpec((1, n_experts), lambda i: (0, 0)),
        ],
        out_specs=[
            pl.BlockSpec((bt, _TOP_K), lambda i: (i, 0)),
            pl.BlockSpec((bt, _TOP_K), lambda i: (i, 0)),
        ],
        out_shape=[
            jax.ShapeDtypeStruct((t, _TOP_K), jnp.int32),
            jax.ShapeDtypeStruct((t, _TOP_K), jnp.float32),
        ],
        compiler_params=pltpu.CompilerParams(
            dimension_semantics=("arbitrary",),
        ),
    )(x2, wt, b2)
    return idx, w


# speedup vs baseline: 1.5019x; 1.5019x over previous
import jax
import jax.numpy as jnp
from jax.experimental import pallas as pl
from jax.experimental.pallas import tpu as pltpu

_N_GROUP = 8
_TOPK_GROUP = 4
_TOP_K = 8
_SCALE = 2.5
_NEG = -1e30


def _gate_kernel(x_ref, wt_ref, b_ref, idx_ref, w_ref):
    x = x_ref[...]
    wt = wt_ref[...]
    logits = jnp.dot(x, wt, preferred_element_type=jnp.float32)
    scores = jax.nn.sigmoid(logits)
    sfc = scores + b_ref[...]

    bt, e = scores.shape
    spg = e // _N_GROUP
    lane = jax.lax.broadcasted_iota(jnp.int32, (bt, e), 1)
    gid = lane // spg

    gs_cols = []
    for g in range(_N_GROUP):
        in_g = gid == g
        m1 = jnp.max(jnp.where(in_g, sfc, _NEG), axis=1, keepdims=True)
        i1 = jnp.min(jnp.where(in_g & (sfc == m1), lane, e), axis=1,
                     keepdims=True)
        m2 = jnp.max(jnp.where(in_g & (lane != i1), sfc, _NEG), axis=1,
                     keepdims=True)
        gs_cols.append(m1 + m2)
    gs = jnp.concatenate(gs_cols, axis=1)

    lane_g = jax.lax.broadcasted_iota(jnp.int32, (bt, _N_GROUP), 1)
    gmask = jnp.zeros((bt, e), dtype=jnp.bool_)
    t = gs
    for _ in range(_TOPK_GROUP):
        m = jnp.max(t, axis=1, keepdims=True)
        gsel = jnp.min(jnp.where(t == m, lane_g, _N_GROUP), axis=1,
                       keepdims=True)
        gmask = gmask | (gid == gsel)
        t = jnp.where(lane_g == gsel, _NEG, t)

    tmp = jnp.where(gmask, sfc, 0.0)

    col = jax.lax.broadcasted_iota(jnp.int32, (bt, _TOP_K), 1)
    acc_idx = jnp.zeros((bt, _TOP_K), dtype=jnp.int32)
    acc_w = jnp.zeros((bt, _TOP_K), dtype=jnp.float32)
    t = tmp
    for k in range(_TOP_K):
        m = jnp.max(t, axis=1, keepdims=True)
        i = jnp.min(jnp.where(t == m, lane, e), axis=1, keepdims=True)
        onehot = lane == i
        wk = jnp.max(jnp.where(onehot, scores, _NEG), axis=1, keepdims=True)
        acc_idx = jnp.where(col == k, i, acc_idx)
        acc_w = jnp.where(col == k, wk, acc_w)
        t = jnp.where(onehot, _NEG, t)

    denom = jnp.sum(acc_w, axis=1, keepdims=True) + 1e-20
    idx_ref[...] = acc_idx
    w_ref[...] = acc_w / denom * _SCALE


def kernel(hidden_states, weight, e_score_correction_bias):
    bsz, seq, h = hidden_states.shape
    n_experts = weight.shape[0]
    t = bsz * seq
    bt = 256

    x2 = hidden_states.reshape(t, h)
    wt = weight.astype(jnp.float32).T
    b2 = e_score_correction_bias.reshape(1, n_experts).astype(jnp.float32)

    idx, w = pl.pallas_call(
        _gate_kernel,
        grid=(t // bt,),
        in_specs=[
            pl.BlockSpec((bt, h), lambda i: (i, 0)),
            pl.BlockSpec((h, n_experts), lambda i: (0, 0)),
            pl.BlockSpec((1, n_experts), lambda i: (0, 0)),
        ],
        out_specs=[
            pl.BlockSpec((bt, _TOP_K), lambda i: (i, 0)),
            pl.BlockSpec((bt, _TOP_K), lambda i: (i, 0)),
        ],
        out_shape=[
            jax.ShapeDtypeStruct((t, _TOP_K), jnp.int32),
            jax.ShapeDtypeStruct((t, _TOP_K), jnp.float32),
        ],
        compiler_params=pltpu.CompilerParams(
            dimension_semantics=("arbitrary",),
        ),
    )(x2, wt, b2)
    return idx, w
